# Initial kernel scaffold; baseline (speedup 1.0000x reference)
import jax
import jax.numpy as jnp
from jax.experimental import pallas as pl


def kernel(x, edge_index, fc1_W, fc1_b, conv1_W, conv1_b, conv2_W, conv2_b, conv3_W, conv3_b, fc2_W, fc2_b):
    raise NotImplementedError("write your pallas kernel here")


# speedup vs baseline: 12.3867x; 12.3867x over previous
import jax
import jax.numpy as jnp
from jax import lax
from jax.experimental import pallas as pl
from jax.experimental.pallas import tpu as pltpu
from jax.experimental.pallas import tpu_sc as plsc

NC = 2
NS = 16
CH = 128
BLK = 1024


def _cdiv(a, b):
  return (a + b - 1) // b


def _sc_deg_kernel(npad, epad):
  mesh = plsc.VectorSubcoreMesh(core_axis_name="c", subcore_axis_name="s")
  rt = npad // (NS * CH)
  outer = epad // (NC * NS * CH * 8)

  def body(dst_hbm, zc_hbm, oc_hbm, out_hbm, acc):
    c = lax.axis_index("c")
    s = lax.axis_index("s")

    def inner(dstbuf, onesb, stage):
      pltpu.sync_copy(zc_hbm, stage)
      pltpu.sync_copy(oc_hbm, onesb)

      @pl.loop(0, rt)
      def _zero(k):
        pltpu.sync_copy(stage, acc.at[pl.ds((s * rt + k) * CH, CH)])

      plsc.subcore_barrier()
      ebase = (c * NS + s) * outer * 8

      @pl.loop(0, outer)
      def _edges(o):
        pltpu.sync_copy(dst_hbm.at[pl.ds(ebase + o * 8, 8)], dstbuf)
        for j in range(8):
          pltpu.sync_copy(onesb, acc.at[dstbuf.at[j]], add=True)

      plsc.subcore_barrier()

      @pl.loop(0, rt)
      def _wout(k):
        r = (s * rt + k) * CH
        pltpu.sync_copy(acc.at[pl.ds(r, CH)], stage)
        pltpu.sync_copy(stage, out_hbm.at[c, pl.ds(r, CH)])

    pl.run_scoped(
        inner,
        pltpu.VMEM((8, CH), jnp.int32),
        pltpu.VMEM((CH, 16), jnp.float32),
        pltpu.VMEM((CH, 16), jnp.float32),
    )

  return pl.kernel(
      body,
      out_type=jax.ShapeDtypeStruct((NC, npad, 16), jnp.float32),
      mesh=mesh,
      scratch_types=[
          pltpu.VMEM_SHARED((npad, 16), jnp.float32),
      ],
      compiler_params=pltpu.CompilerParams(use_tc_tiling_on_sc=False),
  )


def _sc_scatter_kernel(npad, epad):
  mesh = plsc.VectorSubcoreMesh(core_axis_name="c", subcore_axis_name="s")
  rt = npad // (NS * CH)
  outer = epad // (NS * CH * 8)

  def body(y_hbm, src_hbm, dst_hbm, zc_hbm, out_hbm, acc):
    c = lax.axis_index("c")
    s = lax.axis_index("s")

    def inner(srcbuf, dstbuf, rows, stage):
      pltpu.sync_copy(zc_hbm, stage)

      @pl.loop(0, rt)
      def _zero(k):
        pltpu.sync_copy(stage, acc.at[pl.ds((s * rt + k) * CH, CH)])

      plsc.subcore_barrier()
      rbase = s * outer * 8

      @pl.loop(0, outer)
      def _edges(o):
        r0 = rbase + o * 8
        pltpu.sync_copy(src_hbm.at[c, pl.ds(r0, 8)], srcbuf)
        pltpu.sync_copy(dst_hbm.at[pl.ds(r0, 8)], dstbuf)
        for j in range(8):
          pltpu.sync_copy(y_hbm.at[srcbuf.at[j]], rows)
          pltpu.sync_copy(rows, acc.at[dstbuf.at[j]], add=True)

      plsc.subcore_barrier()

      @pl.loop(0, rt)
      def _wout(k):
        r = (s * rt + k) * CH
        pltpu.sync_copy(acc.at[pl.ds(r, CH)], stage)
        pltpu.sync_copy(stage, out_hbm.at[c, pl.ds(r, CH)])

    pl.run_scoped(
        inner,
        pltpu.VMEM((8, CH), jnp.int32),
        pltpu.VMEM((8, CH), jnp.int32),
        pltpu.VMEM((CH, 16), jnp.float32),
        pltpu.VMEM((CH, 16), jnp.float32),
    )

  return pl.kernel(
      body,
      out_type=jax.ShapeDtypeStruct((NC, npad, 16), jnp.float32),
      mesh=mesh,
      scratch_types=[
          pltpu.VMEM_SHARED((npad, 16), jnp.float32),
      ],
      compiler_params=pltpu.CompilerParams(use_tc_tiling_on_sc=False),
  )


_DN = (((1,), (1,)), ((), ()))


def _tc_fc1_kernel(npad):
  nb = npad // BLK

  def body(x_ref, w_ref, b_ref, o_ref):
    h = lax.dot_general(x_ref[...], w_ref[...], _DN,
                        preferred_element_type=jnp.float32)
    o_ref[...] = jnp.maximum(h + b_ref[...], 0.0)

  return pl.pallas_call(
      body,
      grid=(nb,),
      in_specs=[
          pl.BlockSpec((BLK, 4), lambda i: (i, 0)),
          pl.BlockSpec((32, 4), lambda i: (0, 0)),
          pl.BlockSpec((1, 32), lambda i: (0, 0)),
      ],
      out_specs=pl.BlockSpec((BLK, 32), lambda i: (i, 0)),
      out_shape=jax.ShapeDtypeStruct((npad, 32), jnp.float32),
  )


def _tc_scale1_kernel(npad):
  nb = npad // BLK

  def body(dp_ref, h_ref, w_ref, y_ref, dis_ref):
    dp = dp_ref[...]
    deg = dp[0, :, 0] + dp[1, :, 0] + 1.0
    dis = lax.rsqrt(deg)
    yh = lax.dot_general(h_ref[...], w_ref[...], _DN,
                         preferred_element_type=jnp.float32)
    y_ref[0] = yh * dis[:, None]
    dis_ref[...] = dis[:, None]

  return pl.pallas_call(
      body,
      grid=(NC, nb),
      in_specs=[
          pl.BlockSpec((NC, BLK, 16), lambda c, i: (0, i, 0)),
          pl.BlockSpec((BLK, 32), lambda c, i: (i, 0)),
          pl.BlockSpec((16, 32), lambda c, i: (c, 0)),
      ],
      out_specs=[
          pl.BlockSpec((1, BLK, 16), lambda c, i: (c, i, 0)),
          pl.BlockSpec((BLK, 1), lambda c, i: (i, 0)),
      ],
      out_shape=[
          jax.ShapeDtypeStruct((NC, npad, 16), jnp.float32),
          jax.ShapeDtypeStruct((npad, 1), jnp.float32),
      ],
  )


def _tc_layer_kernel(npad):
  nb = npad // BLK

  def body(e_ref, y_ref, dis_ref, b_ref, w_ref, o_ref):
    ev = e_ref[...]
    yv = y_ref[...]
    dis = dis_ref[...]
    b = b_ref[...]
    g0 = jnp.maximum(dis * (ev[0] + yv[0]) + b[:, :16], 0.0)
    g1 = jnp.maximum(dis * (ev[1] + yv[1]) + b[:, 16:], 0.0)
    w = w_ref[...]
    yn = (lax.dot_general(g0, w[:, :16], _DN,
                          preferred_element_type=jnp.float32) +
          lax.dot_general(g1, w[:, 16:], _DN,
                          preferred_element_type=jnp.float32))
    o_ref[0] = yn * dis

  return pl.pallas_call(
      body,
      grid=(NC, nb),
      in_specs=[
          pl.BlockSpec((NC, BLK, 16), lambda c, i: (0, i, 0)),
          pl.BlockSpec((NC, BLK, 16), lambda c, i: (0, i, 0)),
          pl.BlockSpec((BLK, 1), lambda c, i: (i, 0)),
          pl.BlockSpec((1, 32), lambda c, i: (0, 0)),
          pl.BlockSpec((16, 32), lambda c, i: (c, 0)),
      ],
      out_specs=pl.BlockSpec((1, BLK, 16), lambda c, i: (c, i, 0)),
      out_shape=jax.ShapeDtypeStruct((NC, npad, 16), jnp.float32),
  )


def _tc_final_kernel(npad):
  nb = npad // BLK

  def body(e_ref, y_ref, dis_ref, b_ref, w2_ref, b2_ref, o_ref):
    ev = e_ref[...]
    yv = y_ref[...]
    dis = dis_ref[...]
    b = b_ref[...]
    g0 = jnp.maximum(dis * (ev[0] + yv[0]) + b[:, :16], 0.0)
    g1 = jnp.maximum(dis * (ev[1] + yv[1]) + b[:, 16:], 0.0)
    w2 = w2_ref[...]
    out = (lax.dot_general(g0, w2[:, :16], _DN,
                           preferred_element_type=jnp.float32) +
           lax.dot_general(g1, w2[:, 16:], _DN,
                           preferred_element_type=jnp.float32))
    o_ref[...] = out + b2_ref[...]

  return pl.pallas_call(
      body,
      grid=(nb,),
      in_specs=[
          pl.BlockSpec((NC, BLK, 16), lambda i: (0, i, 0)),
          pl.BlockSpec((NC, BLK, 16), lambda i: (0, i, 0)),
          pl.BlockSpec((BLK, 1), lambda i: (i, 0)),
          pl.BlockSpec((1, 32), lambda i: (0, 0)),
          pl.BlockSpec((3, 32), lambda i: (0, 0)),
          pl.BlockSpec((1, 3), lambda i: (0, 0)),
      ],
      out_specs=pl.BlockSpec((BLK, 3), lambda i: (i, 0)),
      out_shape=jax.ShapeDtypeStruct((npad, 3), jnp.float32),
  )


def kernel(x, edge_index, fc1_W, fc1_b, conv1_W, conv1_b, conv2_W, conv2_b,
           conv3_W, conv3_b, fc2_W, fc2_b):
  n = x.shape[0]
  e = edge_index.shape[1]
  npad = NS * CH * _cdiv(n, NS * CH)
  epad = NC * NS * CH * 8 * _cdiv(e, NC * NS * CH * 8)

  src = edge_index[0].astype(jnp.int32)
  dst = edge_index[1].astype(jnp.int32)
  pad_e = epad - e
  src_p = jnp.concatenate([src, jnp.zeros((pad_e,), jnp.int32)])
  dst_p = jnp.concatenate([dst, jnp.full((pad_e,), n, jnp.int32)])
  src2 = jnp.stack([src_p, src_p + npad]).reshape(NC, epad // CH, CH)
  dst2 = dst_p.reshape(epad // CH, CH)
  xp = jnp.zeros((npad, 4), jnp.float32).at[:n].set(x)
  zc = jnp.zeros((CH, 16), jnp.float32)
  oc = jnp.ones((CH, 16), jnp.float32)
  b1 = fc1_b.reshape(1, 32)
  cb1 = conv1_b.reshape(1, 32)
  cb2 = conv2_b.reshape(1, 32)
  cb3 = conv3_b.reshape(1, 32)
  b2 = fc2_b.reshape(1, 3)

  deg_call = _sc_deg_kernel(npad, epad)
  scat_call = _sc_scatter_kernel(npad, epad)
  fc1_call = _tc_fc1_kernel(npad)
  scale1_call = _tc_scale1_kernel(npad)
  layer_call = _tc_layer_kernel(npad)
  final_call = _tc_final_kernel(npad)

  degp = deg_call(dst2, zc, oc)
  h1 = fc1_call(xp, fc1_W, b1)
  y1, dis = scale1_call(degp, h1, conv1_W)
  e1 = scat_call(y1.reshape(NC * npad, 16), src2, dst2, zc)
  y2 = layer_call(e1, y1, dis, cb1, conv2_W)
  e2 = scat_call(y2.reshape(NC * npad, 16), src2, dst2, zc)
  y3 = layer_call(e2, y2, dis, cb2, conv3_W)
  e3 = scat_call(y3.reshape(NC * npad, 16), src2, dst2, zc)
  out = final_call(e3, y3, dis, cb3, fc2_W, b2)
  return out[:n]


# speedup vs baseline: 19.8016x; 1.5986x over previous
import jax
import jax.numpy as jnp
from jax import lax
from jax.experimental import pallas as pl
from jax.experimental.pallas import tpu as pltpu
from jax.experimental.pallas import tpu_sc as plsc

NC = 2
NS = 16
CH = 128
BLK = 1024


def _cdiv(a, b):
  return (a + b - 1) // b


def _sc_deg_kernel(npad, epad):
  mesh = plsc.VectorSubcoreMesh(core_axis_name="c", subcore_axis_name="s")
  rt = npad // (NS * CH)
  outer = epad // (NC * NS * CH * 8)

  def body(dst_hbm, zc_hbm, oc_hbm, out_hbm, acc):
    c = lax.axis_index("c")
    s = lax.axis_index("s")

    def inner(dstbuf, onesb, stage):
      pltpu.sync_copy(zc_hbm, stage)
      pltpu.sync_copy(oc_hbm, onesb)

      @pl.loop(0, rt)
      def _zero(k):
        pltpu.sync_copy(stage, acc.at[pl.ds((s * rt + k) * CH, CH)])

      plsc.subcore_barrier()
      ebase = (c * NS + s) * outer * 8

      @pl.loop(0, outer)
      def _edges(o):
        pltpu.sync_copy(dst_hbm.at[pl.ds(ebase + o * 8, 8)], dstbuf)
        for j in range(8):
          pltpu.sync_copy(onesb, acc.at[dstbuf.at[j]], add=True)

      plsc.subcore_barrier()

      @pl.loop(0, rt)
      def _wout(k):
        r = (s * rt + k) * CH
        pltpu.sync_copy(acc.at[pl.ds(r, CH)], stage)
        pltpu.sync_copy(stage, out_hbm.at[c, pl.ds(r, CH)])

    pl.run_scoped(
        inner,
        pltpu.VMEM((8, CH), jnp.int32),
        pltpu.VMEM((CH, 16), jnp.float32),
        pltpu.VMEM((CH, 16), jnp.float32),
    )

  return pl.kernel(
      body,
      out_type=jax.ShapeDtypeStruct((NC, npad, 16), jnp.float32),
      mesh=mesh,
      scratch_types=[
          pltpu.VMEM_SHARED((npad, 16), jnp.float32),
      ],
      compiler_params=pltpu.CompilerParams(use_tc_tiling_on_sc=False),
  )


def _sc_scatter_kernel(npad, epad):
  mesh = plsc.VectorSubcoreMesh(core_axis_name="c", subcore_axis_name="s")
  rt = npad // (NS * CH)
  outer = epad // (NS * CH * 8)

  def body(y_hbm, src_hbm, dst_hbm, zc_hbm, out_hbm, acc):
    c = lax.axis_index("c")
    s = lax.axis_index("s")

    def inner(srcA, srcB, dstA, dstB, rows, stage, gsem, ssem):
      pltpu.sync_copy(zc_hbm, stage)

      @pl.loop(0, rt)
      def _zero(k):
        pltpu.sync_copy(stage, acc.at[pl.ds((s * rt + k) * CH, CH)])

      plsc.subcore_barrier()
      rbase = s * outer * 8

      def rowslice(b):
        return rows.at[pl.ds(b * CH, CH)]

      def drain_scatters():
        for b in range(8):
          pltpu.make_async_copy(zc_hbm, rowslice(b), ssem.at[b]).wait()

      def do_group(o, srcbuf, dstbuf, first):
        if first:
          @pl.when(o > 0)
          def _():
            drain_scatters()
        else:
          drain_scatters()
        r0 = rbase + o * 8
        pltpu.sync_copy(src_hbm.at[c, pl.ds(r0, 8)], srcbuf)
        pltpu.sync_copy(dst_hbm.at[pl.ds(r0, 8)], dstbuf)
        gathers = []
        for b in range(8):
          gathers.append(
              pltpu.async_copy(y_hbm.at[srcbuf.at[b]], rowslice(b),
                               gsem.at[b]))
        for b in range(8):
          gathers[b].wait()
          pltpu.async_copy(rowslice(b), acc.at[dstbuf.at[b]], ssem.at[b],
                           add=True)

      @pl.loop(0, outer // 2)
      def _grp(t):
        do_group(2 * t, srcA, dstA, True)
        do_group(2 * t + 1, srcB, dstB, False)

      drain_scatters()
      plsc.subcore_barrier()

      @pl.loop(0, rt)
      def _wout(k):
        r = (s * rt + k) * CH
        pltpu.sync_copy(acc.at[pl.ds(r, CH)], stage)
        pltpu.sync_copy(stage, out_hbm.at[c, pl.ds(r, CH)])

    pl.run_scoped(
        inner,
        pltpu.VMEM((8, CH), jnp.int32),
        pltpu.VMEM((8, CH), jnp.int32),
        pltpu.VMEM((8, CH), jnp.int32),
        pltpu.VMEM((8, CH), jnp.int32),
        pltpu.VMEM((8 * CH, 16), jnp.float32),
        pltpu.VMEM((CH, 16), jnp.float32),
        pltpu.SemaphoreType.DMA((8,)),
        pltpu.SemaphoreType.DMA((8,)),
    )

  return pl.kernel(
      body,
      out_type=jax.ShapeDtypeStruct((NC, npad, 16), jnp.float32),
      mesh=mesh,
      scratch_types=[
          pltpu.VMEM_SHARED((npad, 16), jnp.float32),
      ],
      compiler_params=pltpu.CompilerParams(use_tc_tiling_on_sc=False),
  )


_DN = (((1,), (1,)), ((), ()))


def _tc_fc1_kernel(npad):
  nb = npad // BLK

  def body(x_ref, w_ref, b_ref, o_ref):
    h = lax.dot_general(x_ref[...], w_ref[...], _DN,
                        preferred_element_type=jnp.float32)
    o_ref[...] = jnp.maximum(h + b_ref[...], 0.0)

  return pl.pallas_call(
      body,
      grid=(nb,),
      in_specs=[
          pl.BlockSpec((BLK, 4), lambda i: (i, 0)),
          pl.BlockSpec((32, 4), lambda i: (0, 0)),
          pl.BlockSpec((1, 32), lambda i: (0, 0)),
      ],
      out_specs=pl.BlockSpec((BLK, 32), lambda i: (i, 0)),
      out_shape=jax.ShapeDtypeStruct((npad, 32), jnp.float32),
  )


def _tc_scale1_kernel(npad):
  nb = npad // BLK

  def body(dp_ref, h_ref, w_ref, y_ref, dis_ref):
    dp = dp_ref[...]
    deg = dp[0, :, 0] + dp[1, :, 0] + 1.0
    dis = lax.rsqrt(deg)
    yh = lax.dot_general(h_ref[...], w_ref[...], _DN,
                         preferred_element_type=jnp.float32)
    y_ref[0] = yh * dis[:, None]
    dis_ref[...] = dis[:, None]

  return pl.pallas_call(
      body,
      grid=(NC, nb),
      in_specs=[
          pl.BlockSpec((NC, BLK, 16), lambda c, i: (0, i, 0)),
          pl.BlockSpec((BLK, 32), lambda c, i: (i, 0)),
          pl.BlockSpec((16, 32), lambda c, i: (c, 0)),
      ],
      out_specs=[
          pl.BlockSpec((1, BLK, 16), lambda c, i: (c, i, 0)),
          pl.BlockSpec((BLK, 1), lambda c, i: (i, 0)),
      ],
      out_shape=[
          jax.ShapeDtypeStruct((NC, npad, 16), jnp.float32),
          jax.ShapeDtypeStruct((npad, 1), jnp.float32),
      ],
  )


def _tc_layer_kernel(npad):
  nb = npad // BLK

  def body(e_ref, y_ref, dis_ref, b_ref, w_ref, o_ref):
    ev = e_ref[...]
    yv = y_ref[...]
    dis = dis_ref[...]
    b = b_ref[...]
    g0 = jnp.maximum(dis * (ev[0] + yv[0]) + b[:, :16], 0.0)
    g1 = jnp.maximum(dis * (ev[1] + yv[1]) + b[:, 16:], 0.0)
    w = w_ref[...]
    yn = (lax.dot_general(g0, w[:, :16], _DN,
                          preferred_element_type=jnp.float32) +
          lax.dot_general(g1, w[:, 16:], _DN,
                          preferred_element_type=jnp.float32))
    o_ref[0] = yn * dis

  return pl.pallas_call(
      body,
      grid=(NC, nb),
      in_specs=[
          pl.BlockSpec((NC, BLK, 16), lambda c, i: (0, i, 0)),
          pl.BlockSpec((NC, BLK, 16), lambda c, i: (0, i, 0)),
          pl.BlockSpec((BLK, 1), lambda c, i: (i, 0)),
          pl.BlockSpec((1, 32), lambda c, i: (0, 0)),
          pl.BlockSpec((16, 32), lambda c, i: (c, 0)),
      ],
      out_specs=pl.BlockSpec((1, BLK, 16), lambda c, i: (c, i, 0)),
      out_shape=jax.ShapeDtypeStruct((NC, npad, 16), jnp.float32),
  )


def _tc_final_kernel(npad):
  nb = npad // BLK

  def body(e_ref, y_ref, dis_ref, b_ref, w2_ref, b2_ref, o_ref):
    ev = e_ref[...]
    yv = y_ref[...]
    dis = dis_ref[...]
    b = b_ref[...]
    g0 = jnp.maximum(dis * (ev[0] + yv[0]) + b[:, :16], 0.0)
    g1 = jnp.maximum(dis * (ev[1] + yv[1]) + b[:, 16:], 0.0)
    w2 = w2_ref[...]
    out = (lax.dot_general(g0, w2[:, :16], _DN,
                           preferred_element_type=jnp.float32) +
           lax.dot_general(g1, w2[:, 16:], _DN,
                           preferred_element_type=jnp.float32))
    o_ref[...] = out + b2_ref[...]

  return pl.pallas_call(
      body,
      grid=(nb,),
      in_specs=[
          pl.BlockSpec((NC, BLK, 16), lambda i: (0, i, 0)),
          pl.BlockSpec((NC, BLK, 16), lambda i: (0, i, 0)),
          pl.BlockSpec((BLK, 1), lambda i: (i, 0)),
          pl.BlockSpec((1, 32), lambda i: (0, 0)),
          pl.BlockSpec((3, 32), lambda i: (0, 0)),
          pl.BlockSpec((1, 3), lambda i: (0, 0)),
      ],
      out_specs=pl.BlockSpec((BLK, 3), lambda i: (i, 0)),
      out_shape=jax.ShapeDtypeStruct((npad, 3), jnp.float32),
  )


def kernel(x, edge_index, fc1_W, fc1_b, conv1_W, conv1_b, conv2_W, conv2_b,
           conv3_W, conv3_b, fc2_W, fc2_b):
  n = x.shape[0]
  e = edge_index.shape[1]
  npad = NS * CH * _cdiv(n, NS * CH)
  epad = NC * NS * CH * 8 * _cdiv(e, NC * NS * CH * 8)

  src = edge_index[0].astype(jnp.int32)
  dst = edge_index[1].astype(jnp.int32)
  pad_e = epad - e
  src_p = jnp.concatenate([src, jnp.zeros((pad_e,), jnp.int32)])
  dst_p = jnp.concatenate([dst, jnp.full((pad_e,), n, jnp.int32)])
  src2 = jnp.stack([src_p, src_p + npad]).reshape(NC, epad // CH, CH)
  dst2 = dst_p.reshape(epad // CH, CH)
  xp = jnp.zeros((npad, 4), jnp.float32).at[:n].set(x)
  zc = jnp.zeros((CH, 16), jnp.float32)
  oc = jnp.ones((CH, 16), jnp.float32)
  b1 = fc1_b.reshape(1, 32)
  cb1 = conv1_b.reshape(1, 32)
  cb2 = conv2_b.reshape(1, 32)
  cb3 = conv3_b.reshape(1, 32)
  b2 = fc2_b.reshape(1, 3)

  deg_call = _sc_deg_kernel(npad, epad)
  scat_call = _sc_scatter_kernel(npad, epad)
  fc1_call = _tc_fc1_kernel(npad)
  scale1_call = _tc_scale1_kernel(npad)
  layer_call = _tc_layer_kernel(npad)
  final_call = _tc_final_kernel(npad)

  degp = deg_call(dst2, zc, oc)
  h1 = fc1_call(xp, fc1_W, b1)
  y1, dis = scale1_call(degp, h1, conv1_W)
  e1 = scat_call(y1.reshape(NC * npad, 16), src2, dst2, zc)
  y2 = layer_call(e1, y1, dis, cb1, conv2_W)
  e2 = scat_call(y2.reshape(NC * npad, 16), src2, dst2, zc)
  y3 = layer_call(e2, y2, dis, cb2, conv3_W)
  e3 = scat_call(y3.reshape(NC * npad, 16), src2, dst2, zc)
  out = final_call(e3, y3, dis, cb3, fc2_W, b2)
  return out[:n]
